# Initial kernel scaffold; baseline (speedup 1.0000x reference)
import jax
import jax.numpy as jnp
from jax.experimental import pallas as pl


def kernel(idx, logits_table):
    raise NotImplementedError("write your pallas kernel here")


# speedup vs baseline: 1.3849x; 1.3849x over previous
import functools

import jax
import jax.numpy as jnp
from jax import lax
from jax.experimental import pallas as pl
from jax.experimental.pallas import tpu as pltpu
from jax.experimental.pallas import tpu_sc as plsc

VOCAB_SIZE = 1000
N_ROWS = 4096 * 20

_info = plsc.get_sparse_core_info()
NUM_CORES = _info.num_cores
NUM_SUBCORES = _info.num_subcores
NUM_WORKERS = NUM_CORES * NUM_SUBCORES

CHUNK = 64
ROWS_PER_WORKER = N_ROWS // NUM_WORKERS
NUM_CHUNKS = ROWS_PER_WORKER // CHUNK


def _make_gather(V, D):
    mesh = plsc.VectorSubcoreMesh(core_axis_name="c", subcore_axis_name="s")

    @functools.partial(
        pl.kernel,
        mesh=mesh,
        out_type=jax.ShapeDtypeStruct((N_ROWS, D), jnp.float32),
        scratch_types=[
            pltpu.VMEM((CHUNK,), jnp.int32),
            pltpu.VMEM((CHUNK, D), jnp.float32),
            pltpu.SemaphoreType.DMA,
        ],
        compiler_params=pltpu.CompilerParams(use_tc_tiling_on_sc=False),
    )
    def gather_kernel(idx_hbm, table_hbm, out_hbm, idx_v, rows_v, sem):
        wid = lax.axis_index("s") * NUM_CORES + lax.axis_index("c")
        base0 = wid * ROWS_PER_WORKER

        def body(g, carry):
            base = base0 + g * CHUNK
            pltpu.sync_copy(idx_hbm.at[pl.ds(base, CHUNK)], idx_v)
            pltpu.async_copy(table_hbm.at[idx_v], rows_v, sem).wait()
            pltpu.sync_copy(rows_v, out_hbm.at[pl.ds(base, CHUNK)])
            return carry

        lax.fori_loop(0, NUM_CHUNKS, body, 0)

    return gather_kernel


_gather = _make_gather(VOCAB_SIZE, VOCAB_SIZE)


def kernel(idx, logits_table):
    batch, hist = idx.shape
    flat_idx = idx.reshape(-1).astype(jnp.int32)
    out = _gather(flat_idx, logits_table)
    return out.reshape(batch, hist, logits_table.shape[1])


# speedup vs baseline: 1.4102x; 1.0183x over previous
import functools

import jax
import jax.numpy as jnp
from jax import lax
from jax.experimental import pallas as pl
from jax.experimental.pallas import tpu as pltpu
from jax.experimental.pallas import tpu_sc as plsc

VOCAB_SIZE = 1000
N_ROWS = 4096 * 20

_info = plsc.get_sparse_core_info()
NUM_CORES = _info.num_cores
NUM_SUBCORES = _info.num_subcores
NUM_WORKERS = NUM_CORES * NUM_SUBCORES

CHUNK = 64
ROWS_PER_WORKER = N_ROWS // NUM_WORKERS
NUM_CHUNKS = ROWS_PER_WORKER // CHUNK
HALF_STEPS = NUM_CHUNKS // 2


def _make_gather(V, D):
    mesh = plsc.VectorSubcoreMesh(core_axis_name="c", subcore_axis_name="s")

    @functools.partial(
        pl.kernel,
        mesh=mesh,
        out_type=jax.ShapeDtypeStruct((N_ROWS, D), jnp.float32),
        scratch_types=[
            pltpu.VMEM((NUM_CHUNKS, CHUNK), jnp.int32),
            pltpu.VMEM((CHUNK, D), jnp.float32),
            pltpu.VMEM((CHUNK, D), jnp.float32),
            pltpu.SemaphoreType.DMA,
            pltpu.SemaphoreType.DMA,
            pltpu.SemaphoreType.DMA,
            pltpu.SemaphoreType.DMA,
        ],
        compiler_params=pltpu.CompilerParams(use_tc_tiling_on_sc=False),
    )
    def gather_kernel(idx_hbm, table_hbm, out_hbm, idx_v, rows0, rows1,
                      sem_g0, sem_g1, sem_s0, sem_s1):
        cid = lax.axis_index("c")
        sid = lax.axis_index("s")
        wid = sid * NUM_CORES + cid
        base0 = wid * ROWS_PER_WORKER

        pltpu.sync_copy(idx_hbm.at[wid], idx_v)

        def start_gather(g, buf, sem):
            pltpu.async_copy(table_hbm.at[idx_v.at[g]], buf, sem)

        def wait_gather(g, buf, sem):
            pltpu.make_async_copy(table_hbm.at[idx_v.at[g]], buf, sem).wait()

        def start_scatter(g, buf, sem):
            pltpu.async_copy(
                buf, out_hbm.at[pl.ds(base0 + g * CHUNK, CHUNK)], sem)

        def wait_scatter(g, buf, sem):
            pltpu.make_async_copy(
                buf, out_hbm.at[pl.ds(base0 + g * CHUNK, CHUNK)], sem).wait()

        start_gather(0, rows0, sem_g0)
        start_gather(1, rows1, sem_g1)

        def body(h, carry):
            g0 = 2 * h
            g1 = g0 + 1
            wait_gather(g0, rows0, sem_g0)
            start_scatter(g0, rows0, sem_s0)
            wait_gather(g1, rows1, sem_g1)
            start_scatter(g1, rows1, sem_s1)

            @pl.when(h + 1 < HALF_STEPS)
            def _refill():
                wait_scatter(g0, rows0, sem_s0)
                start_gather(g0 + 2, rows0, sem_g0)
                wait_scatter(g1, rows1, sem_s1)
                start_gather(g1 + 2, rows1, sem_g1)

            return carry

        lax.fori_loop(0, HALF_STEPS, body, 0)
        wait_scatter(NUM_CHUNKS - 2, rows0, sem_s0)
        wait_scatter(NUM_CHUNKS - 1, rows1, sem_s1)

    return gather_kernel


_gather = _make_gather(VOCAB_SIZE, VOCAB_SIZE)


def kernel(idx, logits_table):
    batch, hist = idx.shape
    flat_idx = idx.reshape(NUM_WORKERS, NUM_CHUNKS, CHUNK).astype(jnp.int32)
    out = _gather(flat_idx, logits_table)
    return out.reshape(batch, hist, logits_table.shape[1])


# speedup vs baseline: 1.4270x; 1.0119x over previous
import functools

import jax
import jax.numpy as jnp
from jax import lax
from jax.experimental import pallas as pl
from jax.experimental.pallas import tpu as pltpu
from jax.experimental.pallas import tpu_sc as plsc

VOCAB_SIZE = 1000
N_ROWS = 4096 * 20

_info = plsc.get_sparse_core_info()
NUM_CORES = _info.num_cores
NUM_SUBCORES = _info.num_subcores
NUM_WORKERS = NUM_CORES * NUM_SUBCORES

CHUNK = 64
ROWS_PER_WORKER = N_ROWS // NUM_WORKERS
NUM_CHUNKS = ROWS_PER_WORKER // CHUNK
HALF_STEPS = NUM_CHUNKS // 2


def _make_gather(V, D):
    mesh = plsc.VectorSubcoreMesh(core_axis_name="c", subcore_axis_name="s")

    @functools.partial(
        pl.kernel,
        mesh=mesh,
        out_type=jax.ShapeDtypeStruct((N_ROWS * D,), jnp.float32),
        scratch_types=[
            pltpu.VMEM((ROWS_PER_WORKER,), jnp.int32),
            pltpu.VMEM((CHUNK, D), jnp.float32),
            pltpu.VMEM((CHUNK, D), jnp.float32),
            pltpu.SemaphoreType.DMA,
            pltpu.SemaphoreType.DMA,
            pltpu.SemaphoreType.DMA,
            pltpu.SemaphoreType.DMA,
        ],
        compiler_params=pltpu.CompilerParams(use_tc_tiling_on_sc=False),
    )
    def gather_kernel(idx_hbm, table_hbm, out_hbm, idx_v, rows0, rows1,
                      sem_g0, sem_g1, sem_s0, sem_s1):
        cid = lax.axis_index("c")
        sid = lax.axis_index("s")
        wid = sid * NUM_CORES + cid
        base0 = wid * ROWS_PER_WORKER

        pltpu.sync_copy(idx_hbm.at[pl.ds(base0, ROWS_PER_WORKER)], idx_v)

        def start_gather(g, buf, sem):
            pltpu.async_copy(
                table_hbm.at[idx_v.at[pl.ds(g * CHUNK, CHUNK)]], buf, sem)

        def wait_gather(g, buf, sem):
            pltpu.make_async_copy(
                table_hbm.at[idx_v.at[pl.ds(g * CHUNK, CHUNK)]], buf,
                sem).wait()

        def start_scatter(g, buf, sem):
            elem0 = (base0 + g * CHUNK) * D

            def row(r, carry):
                pltpu.async_copy(
                    buf.at[r], out_hbm.at[pl.ds(elem0 + r * D, D)], sem)
                return carry

            lax.fori_loop(0, CHUNK, row, 0)

        def wait_scatter(g, sem):
            sl = out_hbm.at[pl.ds((base0 + g * CHUNK) * D, CHUNK * D)]
            pltpu.make_async_copy(sl, sl, sem).wait()

        start_gather(0, rows0, sem_g0)
        start_gather(1, rows1, sem_g1)

        def body(h, carry):
            g0 = 2 * h
            g1 = g0 + 1
            wait_gather(g0, rows0, sem_g0)
            start_scatter(g0, rows0, sem_s0)
            wait_gather(g1, rows1, sem_g1)
            start_scatter(g1, rows1, sem_s1)

            @pl.when(h + 1 < HALF_STEPS)
            def _refill():
                wait_scatter(g0, sem_s0)
                start_gather(g0 + 2, rows0, sem_g0)
                wait_scatter(g1, sem_s1)
                start_gather(g1 + 2, rows1, sem_g1)

            return carry

        lax.fori_loop(0, HALF_STEPS, body, 0)
        wait_scatter(NUM_CHUNKS - 2, sem_s0)
        wait_scatter(NUM_CHUNKS - 1, sem_s1)

    return gather_kernel


_gather = _make_gather(VOCAB_SIZE, VOCAB_SIZE)


def kernel(idx, logits_table):
    batch, hist = idx.shape
    vocab = logits_table.shape[1]
    flat_idx = idx.reshape(-1).astype(jnp.int32)
    out = _gather(flat_idx, logits_table)
    return out.reshape(batch, hist, vocab)


# speedup vs baseline: 1.4273x; 1.0002x over previous
import functools

import jax
import jax.numpy as jnp
from jax import lax
from jax.experimental import pallas as pl
from jax.experimental.pallas import tpu as pltpu
from jax.experimental.pallas import tpu_sc as plsc

VOCAB_SIZE = 1000
D_PAD = 1024
BATCH = 4096
HIST = 20
N_ROWS = BATCH * HIST

_info = plsc.get_sparse_core_info()
NUM_CORES = _info.num_cores
NUM_SUBCORES = _info.num_subcores
NUM_WORKERS = NUM_CORES * NUM_SUBCORES

CHUNK = 32
ROWS_PER_WORKER = N_ROWS // NUM_WORKERS
NUM_CHUNKS = ROWS_PER_WORKER // CHUNK
HALF_STEPS = NUM_CHUNKS // 2


def _make_gather(V):
    mesh = plsc.VectorSubcoreMesh(core_axis_name="c", subcore_axis_name="s")

    @functools.partial(
        pl.kernel,
        mesh=mesh,
        out_type=jax.ShapeDtypeStruct((N_ROWS, D_PAD), jnp.float32),
        scratch_types=[
            pltpu.VMEM((ROWS_PER_WORKER,), jnp.int32),
            pltpu.VMEM((CHUNK, D_PAD), jnp.float32),
            pltpu.VMEM((CHUNK, D_PAD), jnp.float32),
            pltpu.SemaphoreType.DMA,
            pltpu.SemaphoreType.DMA,
            pltpu.SemaphoreType.DMA,
            pltpu.SemaphoreType.DMA,
        ],
    )
    def gather_kernel(idx_hbm, table_hbm, out_hbm, idx_v, rows0, rows1,
                      sem_g0, sem_g1, sem_s0, sem_s1):
        cid = lax.axis_index("c")
        sid = lax.axis_index("s")
        wid = sid * NUM_CORES + cid
        base0 = wid * ROWS_PER_WORKER

        pltpu.sync_copy(idx_hbm.at[pl.ds(base0, ROWS_PER_WORKER)], idx_v)

        def start_gather(g, buf, sem):
            pltpu.async_copy(
                table_hbm.at[idx_v.at[pl.ds(g * CHUNK, CHUNK)]], buf, sem)

        def wait_gather(g, buf, sem):
            pltpu.make_async_copy(
                table_hbm.at[idx_v.at[pl.ds(g * CHUNK, CHUNK)]], buf,
                sem).wait()

        def start_scatter(g, buf, sem):
            pltpu.async_copy(
                buf, out_hbm.at[pl.ds(base0 + g * CHUNK, CHUNK)], sem)

        def wait_scatter(g, buf, sem):
            pltpu.make_async_copy(
                buf, out_hbm.at[pl.ds(base0 + g * CHUNK, CHUNK)], sem).wait()

        start_gather(0, rows0, sem_g0)
        start_gather(1, rows1, sem_g1)

        def body(h, carry):
            g0 = 2 * h
            g1 = g0 + 1
            wait_gather(g0, rows0, sem_g0)
            start_scatter(g0, rows0, sem_s0)
            wait_gather(g1, rows1, sem_g1)
            start_scatter(g1, rows1, sem_s1)

            @pl.when(h + 1 < HALF_STEPS)
            def _refill():
                wait_scatter(g0, rows0, sem_s0)
                start_gather(g0 + 2, rows0, sem_g0)
                wait_scatter(g1, rows1, sem_s1)
                start_gather(g1 + 2, rows1, sem_g1)

            return carry

        lax.fori_loop(0, HALF_STEPS, body, 0)
        wait_scatter(NUM_CHUNKS - 2, rows0, sem_s0)
        wait_scatter(NUM_CHUNKS - 1, rows1, sem_s1)

    return gather_kernel


_gather = _make_gather(VOCAB_SIZE)

RELAYOUT_B = 32


def _relayout_body(in_ref, out_ref):
    x = in_ref[:, :VOCAB_SIZE]
    out_ref[...] = x.reshape(RELAYOUT_B, HIST, VOCAB_SIZE)


def _relayout(rows_padded):
    return pl.pallas_call(
        _relayout_body,
        grid=(BATCH // RELAYOUT_B,),
        in_specs=[
            pl.BlockSpec((RELAYOUT_B * HIST, D_PAD), lambda i: (i, 0))
        ],
        out_specs=pl.BlockSpec(
            (RELAYOUT_B, HIST, VOCAB_SIZE), lambda i: (i, 0, 0)),
        out_shape=jax.ShapeDtypeStruct(
            (BATCH, HIST, VOCAB_SIZE), jnp.float32),
    )(rows_padded)


def kernel(idx, logits_table):
    flat_idx = idx.reshape(-1).astype(jnp.int32)
    table_pad = jnp.pad(logits_table, ((0, 0), (0, D_PAD - VOCAB_SIZE)))
    rows_padded = _gather(flat_idx, table_pad)
    return _relayout(rows_padded)


# speedup vs baseline: 1.4745x; 1.0331x over previous
import functools

import jax
import jax.numpy as jnp
from jax import lax
from jax.experimental import pallas as pl
from jax.experimental.pallas import tpu as pltpu
from jax.experimental.pallas import tpu_sc as plsc

VOCAB = 1000
BATCH = 4096
HIST = 20
HIST_PAD = 24
LANES = 16

_info = plsc.get_sparse_core_info()
NUM_CORES = _info.num_cores
NUM_SUBCORES = _info.num_subcores
NUM_WORKERS = NUM_CORES * NUM_SUBCORES

B_BLOCK = BATCH // NUM_WORKERS
V_OCTETS = VOCAB // 8
VP_STEPS = (V_OCTETS + 1) // 2


def _make_tgather():
    mesh = plsc.VectorSubcoreMesh(core_axis_name="c", subcore_axis_name="s")

    @functools.partial(
        pl.kernel,
        mesh=mesh,
        out_type=jax.ShapeDtypeStruct((HIST, VOCAB, BATCH), jnp.float32),
        scratch_types=[
            pltpu.VMEM((HIST_PAD, B_BLOCK), jnp.int32),
            pltpu.VMEM((8 * VOCAB,), jnp.float32),
            pltpu.VMEM((8 * VOCAB,), jnp.float32),
            pltpu.VMEM((8, B_BLOCK), jnp.float32),
            pltpu.VMEM((8, B_BLOCK), jnp.float32),
            pltpu.SemaphoreType.DMA,
            pltpu.SemaphoreType.DMA,
            pltpu.SemaphoreType.DMA,
            pltpu.SemaphoreType.DMA,
        ],
        compiler_params=pltpu.CompilerParams(needs_layout_passes=False),
    )
    def tgather_kernel(idx_hbm, tab_hbm, out_hbm, idx_v, st0, st1,
                       ob0, ob1, sem_t0, sem_t1, sem_o0, sem_o1):
        cid = lax.axis_index("c")
        sid = lax.axis_index("s")
        wid = sid * NUM_CORES + cid
        bcol = wid * B_BLOCK

        pltpu.sync_copy(idx_hbm.at[:, pl.ds(bcol, B_BLOCK)], idx_v)

        def start_stage(vo, st, sem):
            pltpu.async_copy(tab_hbm.at[pl.ds(vo * 8 * VOCAB, 8 * VOCAB)],
                             st, sem)

        def wait_stage(vo, st, sem):
            pltpu.make_async_copy(
                tab_hbm.at[pl.ds(vo * 8 * VOCAB, 8 * VOCAB)], st, sem).wait()

        def out_dst(h, vo):
            return out_hbm.at[h, pl.ds(vo * 8, 8), pl.ds(bcol, B_BLOCK)]

        def fill_and_send(vo, h, st, ob, sem, is_first):
            @pl.when(jnp.logical_not(is_first))
            def _():
                pltpu.make_async_copy(ob, out_dst(h, vo), sem).wait()

            for lb in range(B_BLOCK // LANES):
                iv = idx_v[h, pl.ds(lb * LANES, LANES)]
                for s in range(8):
                    vals = plsc.load_gather(st, [iv + (s * VOCAB)])
                    ob[s, pl.ds(lb * LANES, LANES)] = vals
            pltpu.async_copy(ob, out_dst(h, vo), sem)

        def compute_octet(vo, st, is_first_octet):
            def hbody(hp, carry):
                h0 = 2 * hp
                first = jnp.logical_and(is_first_octet, hp == 0)
                fill_and_send(vo, h0, st, ob0, sem_o0, first)
                fill_and_send(vo, h0 + 1, st, ob1, sem_o1, first)
                return carry
            lax.fori_loop(0, HIST // 2, hbody, 0)

        start_stage(0, st0, sem_t0)

        def vbody(vp, carry):
            v0 = 2 * vp
            v1 = v0 + 1

            @pl.when(v1 < V_OCTETS)
            def _():
                start_stage(v1, st1, sem_t1)

            wait_stage(v0, st0, sem_t0)
            compute_octet(v0, st0, v0 == 0)

            @pl.when(v1 < V_OCTETS)
            def _():
                @pl.when(v0 + 2 < V_OCTETS)
                def _():
                    start_stage(v0 + 2, st0, sem_t0)

                wait_stage(v1, st1, sem_t1)
                compute_octet(v1, st1, False)

            return carry

        lax.fori_loop(0, VP_STEPS, vbody, 0)

        pltpu.make_async_copy(ob0, out_dst(HIST - 2, V_OCTETS - 1),
                              sem_o0).wait()
        pltpu.make_async_copy(ob1, out_dst(HIST - 1, V_OCTETS - 1),
                              sem_o1).wait()

    return tgather_kernel


_tgather = _make_tgather()


def kernel(idx, logits_table):
    idx_tp = jnp.pad(idx.T.astype(jnp.int32),
                     ((0, HIST_PAD - HIST), (0, 0)))
    tab_t = logits_table.T.reshape(-1)
    out_t = _tgather(idx_tp, tab_t)
    return jnp.transpose(out_t, (2, 0, 1))


# speedup vs baseline: 1.8402x; 1.2480x over previous
import functools

import jax
import jax.numpy as jnp
from jax import lax
from jax.experimental import pallas as pl
from jax.experimental.pallas import tpu as pltpu
from jax.experimental.pallas import tpu_sc as plsc

VOCAB = 1000
BATCH = 4096
HIST = 20
HIST_PAD = 24
LANES = 16

_info = plsc.get_sparse_core_info()
NUM_CORES = _info.num_cores
NUM_SUBCORES = _info.num_subcores
NUM_WORKERS = NUM_CORES * NUM_SUBCORES

B_BLOCK = BATCH // NUM_WORKERS
V_OCTETS = VOCAB // 8
VP_STEPS = (V_OCTETS + 1) // 2


def _make_tgather():
    mesh = plsc.VectorSubcoreMesh(core_axis_name="c", subcore_axis_name="s")

    @functools.partial(
        pl.kernel,
        mesh=mesh,
        out_type=jax.ShapeDtypeStruct((HIST, VOCAB, BATCH), jnp.float32),
        scratch_types=[
            pltpu.VMEM((HIST_PAD, B_BLOCK), jnp.int32),
            pltpu.VMEM((8 * VOCAB,), jnp.float32),
            pltpu.VMEM((8 * VOCAB,), jnp.float32),
            pltpu.VMEM((8, B_BLOCK), jnp.float32),
            pltpu.VMEM((8, B_BLOCK), jnp.float32),
            pltpu.SemaphoreType.DMA,
            pltpu.SemaphoreType.DMA,
            pltpu.SemaphoreType.DMA,
            pltpu.SemaphoreType.DMA,
        ],
        compiler_params=pltpu.CompilerParams(needs_layout_passes=False),
    )
    def tgather_kernel(idx_hbm, tab_hbm, out_hbm, idx_v, st0, st1,
                       ob0, ob1, sem_t0, sem_t1, sem_o0, sem_o1):
        cid = lax.axis_index("c")
        sid = lax.axis_index("s")
        wid = sid * NUM_CORES + cid
        bcol = wid * B_BLOCK

        pltpu.sync_copy(idx_hbm.at[:, pl.ds(bcol, B_BLOCK)], idx_v)

        def start_stage(vo, st, sem):
            pltpu.async_copy(tab_hbm.at[pl.ds(vo * 8 * VOCAB, 8 * VOCAB)],
                             st, sem)

        def wait_stage(vo, st, sem):
            pltpu.make_async_copy(
                tab_hbm.at[pl.ds(vo * 8 * VOCAB, 8 * VOCAB)], st, sem).wait()

        def out_dst(h, vo):
            return out_hbm.at[h, pl.ds(vo * 8, 8), pl.ds(bcol, B_BLOCK)]

        def fill_pair(vo, h0, st, is_first):
            h1 = h0 + 1

            @pl.when(jnp.logical_not(is_first))
            def _():
                pltpu.make_async_copy(ob0, out_dst(h0, vo), sem_o0).wait()
                pltpu.make_async_copy(ob1, out_dst(h1, vo), sem_o1).wait()

            for lb in range(B_BLOCK // LANES):
                sl = pl.ds(lb * LANES, LANES)
                iv0 = idx_v[h0, sl]
                iv1 = idx_v[h1, sl]
                for s in range(8):
                    v0 = plsc.load_gather(st, [iv0 + (s * VOCAB)])
                    v1 = plsc.load_gather(st, [iv1 + (s * VOCAB)])
                    ob0[s, sl] = v0
                    ob1[s, sl] = v1
            pltpu.async_copy(ob0, out_dst(h0, vo), sem_o0)
            pltpu.async_copy(ob1, out_dst(h1, vo), sem_o1)

        def compute_octet(vo, st, is_first_octet):
            def hbody(hp, carry):
                first = jnp.logical_and(is_first_octet, hp == 0)
                fill_pair(vo, 2 * hp, st, first)
                return carry
            lax.fori_loop(0, HIST // 2, hbody, 0)

        start_stage(0, st0, sem_t0)

        def vbody(vp, carry):
            v0 = 2 * vp
            v1 = v0 + 1

            @pl.when(v1 < V_OCTETS)
            def _():
                start_stage(v1, st1, sem_t1)

            wait_stage(v0, st0, sem_t0)
            compute_octet(v0, st0, v0 == 0)

            @pl.when(v1 < V_OCTETS)
            def _():
                @pl.when(v0 + 2 < V_OCTETS)
                def _():
                    start_stage(v0 + 2, st0, sem_t0)

                wait_stage(v1, st1, sem_t1)
                compute_octet(v1, st1, False)

            return carry

        lax.fori_loop(0, VP_STEPS, vbody, 0)

        pltpu.make_async_copy(ob0, out_dst(HIST - 2, V_OCTETS - 1),
                              sem_o0).wait()
        pltpu.make_async_copy(ob1, out_dst(HIST - 1, V_OCTETS - 1),
                              sem_o1).wait()

    return tgather_kernel


_tgather = _make_tgather()


def kernel(idx, logits_table):
    idx_tp = jnp.pad(idx.T.astype(jnp.int32),
                     ((0, HIST_PAD - HIST), (0, 0)))
    tab_t = logits_table.T.reshape(-1)
    out_t = _tgather(idx_tp, tab_t)
    return jnp.transpose(out_t, (2, 0, 1))


# speedup vs baseline: 3.3116x; 1.7995x over previous
import functools

import jax
import jax.numpy as jnp
from jax import lax
from jax.experimental import pallas as pl
from jax.experimental.pallas import tpu as pltpu
from jax.experimental.pallas import tpu_sc as plsc

VOCAB = 1000
BATCH = 4096
HIST = 20
HIST_PAD = 24
LANES = 16

_info = plsc.get_sparse_core_info()
NUM_CORES = _info.num_cores
NUM_SUBCORES = _info.num_subcores
NUM_WORKERS = NUM_CORES * NUM_SUBCORES

B_BLOCK = BATCH // NUM_WORKERS
V_OCTETS = VOCAB // 8
VP_STEPS = (V_OCTETS + 1) // 2


def _make_tgather():
    mesh = plsc.VectorSubcoreMesh(core_axis_name="c", subcore_axis_name="s")

    @functools.partial(
        pl.kernel,
        mesh=mesh,
        out_type=jax.ShapeDtypeStruct((HIST, VOCAB, BATCH), jnp.float32),
        scratch_types=[
            pltpu.VMEM((HIST_PAD, B_BLOCK), jnp.int32),
            pltpu.VMEM((8 * VOCAB,), jnp.float32),
            pltpu.VMEM((8 * VOCAB,), jnp.float32),
            pltpu.VMEM((8, B_BLOCK), jnp.float32),
            pltpu.VMEM((8, B_BLOCK), jnp.float32),
            pltpu.SemaphoreType.DMA,
            pltpu.SemaphoreType.DMA,
            pltpu.SemaphoreType.DMA,
            pltpu.SemaphoreType.DMA,
        ],
        compiler_params=pltpu.CompilerParams(needs_layout_passes=False),
    )
    def tgather_kernel(idx_hbm, tab_hbm, out_hbm, idx_v, st0, st1,
                       ob0, ob1, sem_t0, sem_t1, sem_o0, sem_o1):
        cid = lax.axis_index("c")
        sid = lax.axis_index("s")
        wid = sid * NUM_CORES + cid
        bcol = wid * B_BLOCK

        pltpu.sync_copy(idx_hbm.at[:, pl.ds(bcol, B_BLOCK)], idx_v)

        def start_stage(vo, st, sem):
            pltpu.async_copy(tab_hbm.at[pl.ds(vo * 8 * VOCAB, 8 * VOCAB)],
                             st, sem)

        def wait_stage(vo, st, sem):
            pltpu.make_async_copy(
                tab_hbm.at[pl.ds(vo * 8 * VOCAB, 8 * VOCAB)], st, sem).wait()

        def out_dst(h, vo):
            return out_hbm.at[h, pl.ds(vo * 8, 8), pl.ds(bcol, B_BLOCK)]

        def fill_pair(vo, h0, st, is_first):
            h1 = h0 + 1

            @pl.when(jnp.logical_not(is_first))
            def _():
                pltpu.make_async_copy(ob0, out_dst(h0, vo), sem_o0).wait()
                pltpu.make_async_copy(ob1, out_dst(h1, vo), sem_o1).wait()

            @plsc.parallel_loop(0, B_BLOCK // LANES, unroll=B_BLOCK // LANES)
            def _(lb):
                sl = pl.ds(lb * LANES, LANES)
                iv0 = idx_v[h0, sl]
                iv1 = idx_v[h1, sl]
                for s in range(8):
                    v0 = plsc.load_gather(st, [iv0 + (s * VOCAB)])
                    v1 = plsc.load_gather(st, [iv1 + (s * VOCAB)])
                    ob0[s, sl] = v0
                    ob1[s, sl] = v1
            pltpu.async_copy(ob0, out_dst(h0, vo), sem_o0)
            pltpu.async_copy(ob1, out_dst(h1, vo), sem_o1)

        def compute_octet(vo, st, is_first_octet):
            def hbody(hp, carry):
                first = jnp.logical_and(is_first_octet, hp == 0)
                fill_pair(vo, 2 * hp, st, first)
                return carry
            lax.fori_loop(0, HIST // 2, hbody, 0)

        start_stage(0, st0, sem_t0)

        def vbody(vp, carry):
            v0 = 2 * vp
            v1 = v0 + 1

            @pl.when(v1 < V_OCTETS)
            def _():
                start_stage(v1, st1, sem_t1)

            wait_stage(v0, st0, sem_t0)
            compute_octet(v0, st0, v0 == 0)

            @pl.when(v1 < V_OCTETS)
            def _():
                @pl.when(v0 + 2 < V_OCTETS)
                def _():
                    start_stage(v0 + 2, st0, sem_t0)

                wait_stage(v1, st1, sem_t1)
                compute_octet(v1, st1, False)

            return carry

        lax.fori_loop(0, VP_STEPS, vbody, 0)

        pltpu.make_async_copy(ob0, out_dst(HIST - 2, V_OCTETS - 1),
                              sem_o0).wait()
        pltpu.make_async_copy(ob1, out_dst(HIST - 1, V_OCTETS - 1),
                              sem_o1).wait()

    return tgather_kernel


_tgather = _make_tgather()


def kernel(idx, logits_table):
    idx_tp = jnp.pad(idx.T.astype(jnp.int32),
                     ((0, HIST_PAD - HIST), (0, 0)))
    tab_t = logits_table.T.reshape(-1)
    out_t = _tgather(idx_tp, tab_t)
    return jnp.transpose(out_t, (2, 0, 1))


# speedup vs baseline: 4.4967x; 1.3579x over previous
import functools

import jax
import jax.numpy as jnp
from jax import lax
from jax.experimental import pallas as pl
from jax.experimental.pallas import tpu as pltpu
from jax.experimental.pallas import tpu_sc as plsc

VOCAB = 1000
BATCH = 4096
HIST = 20
HIST_PAD = 24
LANES = 16

_info = plsc.get_sparse_core_info()
NUM_CORES = _info.num_cores
NUM_SUBCORES = _info.num_subcores
NUM_WORKERS = NUM_CORES * NUM_SUBCORES

B_BLOCK = BATCH // NUM_WORKERS
V_OCTETS = VOCAB // 8
VP_STEPS = (V_OCTETS + 1) // 2


def _make_tgather():
    mesh = plsc.VectorSubcoreMesh(core_axis_name="c", subcore_axis_name="s")

    @functools.partial(
        pl.kernel,
        mesh=mesh,
        out_type=jax.ShapeDtypeStruct((HIST, VOCAB, BATCH), jnp.float32),
        scratch_types=[
            pltpu.VMEM((HIST_PAD, B_BLOCK), jnp.int32),
            pltpu.VMEM((8 * VOCAB,), jnp.float32),
            pltpu.VMEM((8 * VOCAB,), jnp.float32),
            pltpu.VMEM((8, B_BLOCK), jnp.float32),
            pltpu.VMEM((8, B_BLOCK), jnp.float32),
            pltpu.VMEM((8, B_BLOCK), jnp.float32),
            pltpu.VMEM((8, B_BLOCK), jnp.float32),
            pltpu.SemaphoreType.DMA,
            pltpu.SemaphoreType.DMA,
            pltpu.SemaphoreType.DMA,
            pltpu.SemaphoreType.DMA,
            pltpu.SemaphoreType.DMA,
            pltpu.SemaphoreType.DMA,
        ],
        compiler_params=pltpu.CompilerParams(needs_layout_passes=False),
    )
    def tgather_kernel(idx_hbm, tab_hbm, out_hbm, idx_v, st0, st1,
                       ob0, ob1, ob2, ob3, sem_t0, sem_t1,
                       sem_o0, sem_o1, sem_o2, sem_o3):
        cid = lax.axis_index("c")
        sid = lax.axis_index("s")
        wid = sid * NUM_CORES + cid
        bcol = wid * B_BLOCK

        pltpu.sync_copy(idx_hbm.at[:, pl.ds(bcol, B_BLOCK)], idx_v)

        def start_stage(vo, st, sem):
            pltpu.async_copy(tab_hbm.at[pl.ds(vo * 8 * VOCAB, 8 * VOCAB)],
                             st, sem)

        def wait_stage(vo, st, sem):
            pltpu.make_async_copy(
                tab_hbm.at[pl.ds(vo * 8 * VOCAB, 8 * VOCAB)], st, sem).wait()

        def out_dst(h, vo):
            return out_hbm.at[h, pl.ds(vo * 8, 8), pl.ds(bcol, B_BLOCK)]

        def fill_pair(vo, h0, st, oba, obb, sema, semb, is_first):
            h1 = h0 + 1

            @pl.when(jnp.logical_not(is_first))
            def _():
                pltpu.make_async_copy(oba, out_dst(h0, vo), sema).wait()
                pltpu.make_async_copy(obb, out_dst(h1, vo), semb).wait()

            @plsc.parallel_loop(0, B_BLOCK // LANES, unroll=B_BLOCK // LANES)
            def _(lb):
                sl = pl.ds(lb * LANES, LANES)
                iv0 = idx_v[h0, sl]
                iv1 = idx_v[h1, sl]
                for s in range(8):
                    v0 = plsc.load_gather(st, [iv0 + (s * VOCAB)])
                    v1 = plsc.load_gather(st, [iv1 + (s * VOCAB)])
                    oba[s, sl] = v0
                    obb[s, sl] = v1
            pltpu.async_copy(oba, out_dst(h0, vo), sema)
            pltpu.async_copy(obb, out_dst(h1, vo), semb)

        def compute_octet(vo, st, is_first_octet):
            def hbody(hq, carry):
                first = jnp.logical_and(is_first_octet, hq == 0)
                fill_pair(vo, 4 * hq, st, ob0, ob1, sem_o0, sem_o1, first)
                fill_pair(vo, 4 * hq + 2, st, ob2, ob3, sem_o2, sem_o3,
                          first)
                return carry
            lax.fori_loop(0, HIST // 4, hbody, 0)

        start_stage(0, st0, sem_t0)

        def vbody(vp, carry):
            v0 = 2 * vp
            v1 = v0 + 1

            @pl.when(v1 < V_OCTETS)
            def _():
                start_stage(v1, st1, sem_t1)

            wait_stage(v0, st0, sem_t0)
            compute_octet(v0, st0, v0 == 0)

            @pl.when(v1 < V_OCTETS)
            def _():
                @pl.when(v0 + 2 < V_OCTETS)
                def _():
                    start_stage(v0 + 2, st0, sem_t0)

                wait_stage(v1, st1, sem_t1)
                compute_octet(v1, st1, False)

            return carry

        lax.fori_loop(0, VP_STEPS, vbody, 0)

        pltpu.make_async_copy(ob0, out_dst(HIST - 4, V_OCTETS - 1),
                              sem_o0).wait()
        pltpu.make_async_copy(ob1, out_dst(HIST - 3, V_OCTETS - 1),
                              sem_o1).wait()
        pltpu.make_async_copy(ob2, out_dst(HIST - 2, V_OCTETS - 1),
                              sem_o2).wait()
        pltpu.make_async_copy(ob3, out_dst(HIST - 1, V_OCTETS - 1),
                              sem_o3).wait()

    return tgather_kernel


_tgather = _make_tgather()


def kernel(idx, logits_table):
    idx_tp = jnp.pad(idx.T.astype(jnp.int32),
                     ((0, HIST_PAD - HIST), (0, 0)))
    tab_t = logits_table.T.reshape(-1)
    out_t = _tgather(idx_tp, tab_t)
    return jnp.transpose(out_t, (2, 0, 1))
